# baseline (speedup 1.0000x reference)
import functools

import jax
import jax.numpy as jnp
from jax import lax
from jax.experimental import pallas as pl
from jax.experimental.pallas import tpu as pltpu
from jax.experimental.pallas import tpu_sc as plsc

_E = 8
_K = 2
_BT = 256
_CH = 512

_SC_INFO = plsc.get_sparse_core_info()
_NW = _SC_INFO.num_cores * _SC_INFO.num_subcores
_L = _SC_INFO.num_lanes


def _gate_kernel(x_ref, gw_ref, gb_ref, prob_ref, pos1_ref, pos2_ref,
                 w1n_ref, w2n_ref, te_ref, act_ref, nt):
    t = x_ref.shape[0]
    logits = jnp.dot(x_ref[...], gw_ref[...],
                     preferred_element_type=jnp.float32) + gb_ref[...]
    m = jnp.max(logits, axis=1, keepdims=True)
    p = jnp.exp(logits - m)
    prob = p / jnp.sum(p, axis=1, keepdims=True)
    prob_ref[...] = prob

    iota_e = lax.broadcasted_iota(jnp.int32, (t, _E), 1)
    m1 = jnp.max(prob, axis=1, keepdims=True)
    i1 = jnp.min(jnp.where(prob == m1, iota_e, _E), axis=1, keepdims=True)
    masked = jnp.where(iota_e == i1, -1.0, prob)
    m2 = jnp.max(masked, axis=1, keepdims=True)
    i2 = jnp.min(jnp.where(masked == m2, iota_e, _E), axis=1, keepdims=True)

    e21 = jnp.exp(m2 - m1)
    w1n_ref[...] = jnp.broadcast_to(1.0 / (1.0 + e21), w1n_ref.shape)
    w2n_ref[...] = jnp.broadcast_to(e21 / (1.0 + e21), w2n_ref.shape)

    oh1 = (iota_e == i1).astype(jnp.float32)
    oh2 = (iota_e == i2).astype(jnp.float32)
    rr = lax.broadcasted_iota(jnp.int32, (_CH, _CH), 0)
    cc = lax.broadcasted_iota(jnp.int32, (_CH, _CH), 1)
    ltri = (cc < rr).astype(jnp.float32)

    base = jnp.zeros((1, _E), jnp.float32)
    ranks = []
    for oh in (oh1, oh2):
        for c in range(t // _CH):
            blk = oh[c * _CH:(c + 1) * _CH]
            cum = jnp.dot(ltri, blk, preferred_element_type=jnp.float32) + base
            ranks.append(cum)
            base = base + jnp.sum(blk, axis=0, keepdims=True)
    rank1 = jnp.concatenate(ranks[: t // _CH], axis=0)
    rank2 = jnp.concatenate(ranks[t // _CH:], axis=0)

    counts = base
    padded = jnp.ceil(counts / _BT) * _BT
    er = lax.broadcasted_iota(jnp.int32, (_E, _E), 0)
    ec = lax.broadcasted_iota(jnp.int32, (_E, _E), 1)
    u8 = (er < ec).astype(jnp.float32)
    pad_off = jnp.dot(padded, u8, preferred_element_type=jnp.float32)

    pos1 = jnp.sum((pad_off + rank1) * oh1, axis=1, keepdims=True)
    pos2 = jnp.sum((pad_off + rank2) * oh2, axis=1, keepdims=True)
    pos1_ref[...] = pos1.astype(jnp.int32)
    pos2_ref[...] = pos2.astype(jnp.int32)

    pad_end = pad_off + padded
    ts = lax.broadcasted_iota(jnp.int32, (nt, 1), 0).astype(jnp.float32) * _BT
    te = jnp.sum((pad_end <= ts).astype(jnp.int32), axis=1, keepdims=True)
    te = jnp.minimum(te, _E - 1)
    te_ref[...] = te
    tile_iota = lax.broadcasted_iota(jnp.int32, (nt, _E), 1)
    pe = jnp.sum(jnp.where(tile_iota == te, pad_end, 0.0), axis=1,
                 keepdims=True)
    act_ref[...] = (ts < pe).astype(jnp.int32)


def _gating(xf, gate_W, gate_b, nt):
    t = xf.shape[0]
    f32, i32 = jnp.float32, jnp.int32
    out_shape = (
        jax.ShapeDtypeStruct((t, _E), f32),
        jax.ShapeDtypeStruct((t, 1), i32),
        jax.ShapeDtypeStruct((t, 1), i32),
        jax.ShapeDtypeStruct((t, _L), f32),
        jax.ShapeDtypeStruct((t, _L), f32),
        jax.ShapeDtypeStruct((nt, 1), i32),
        jax.ShapeDtypeStruct((nt, 1), i32),
    )
    return pl.pallas_call(
        functools.partial(_gate_kernel, nt=nt),
        out_shape=out_shape,
    )(xf, gate_W, gate_b.reshape(1, _E))


def _ffn_kernel(te_ref, act_ref, xs_ref, w1_ref, b1_ref, w2_ref, b2_ref,
                out_ref):
    i = pl.program_id(0)
    e = te_ref[i]

    @pl.when(act_ref[i] != 0)
    def _():
        x = xs_ref[...]
        h = jnp.dot(x, w1_ref[0], preferred_element_type=jnp.float32)
        h = jnp.maximum(h + b1_ref[e][None, :], 0.0)
        y = jnp.dot(h, w2_ref[0], preferred_element_type=jnp.float32)
        out_ref[...] = y + b2_ref[e][None, :]


def _grouped_ffn(xs, tile_expert, act, W1, b1, W2, b2, nt, d):
    grid_spec = pltpu.PrefetchScalarGridSpec(
        num_scalar_prefetch=2,
        grid=(nt,),
        in_specs=[
            pl.BlockSpec((_BT, d), lambda i, te, act: (act[i] * i, 0)),
            pl.BlockSpec((1, d, d), lambda i, te, act: (te[i], 0, 0)),
            pl.BlockSpec((_E, d), lambda i, te, act: (0, 0)),
            pl.BlockSpec((1, d, d), lambda i, te, act: (te[i], 0, 0)),
            pl.BlockSpec((_E, d), lambda i, te, act: (0, 0)),
        ],
        out_specs=pl.BlockSpec(
            (_BT, d), lambda i, te, act: (jnp.where(act[i] != 0, i, nt), 0)),
    )
    return pl.pallas_call(
        _ffn_kernel,
        grid_spec=grid_spec,
        out_shape=jax.ShapeDtypeStruct(((nt + 1) * _BT, d), jnp.float32),
    )(tile_expert, act, xs, W1, b1, W2, b2)


def _make_dispatch(t, d, ntot):
    tpw = t // _NW
    mesh = plsc.VectorSubcoreMesh(core_axis_name="c", subcore_axis_name="s")

    nch = 2
    cs = tpw // nch

    @functools.partial(
        pl.kernel,
        mesh=mesh,
        out_type=jax.ShapeDtypeStruct((ntot, d), jnp.float32),
        scratch_types=[
            pltpu.VMEM((nch, cs), jnp.int32),
            pltpu.VMEM((nch, cs), jnp.int32),
            pltpu.VMEM((nch, cs, d), jnp.float32),
            pltpu.SemaphoreType.DMA,
        ]
        + [pltpu.SemaphoreType.DMA] * (2 * nch),
    )
    def disp(x_hbm, p1_hbm, p2_hbm, xs_hbm, p1_v, p2_v, rows_v, rsem, *sems):
        wid = lax.axis_index("s") * _SC_INFO.num_cores + lax.axis_index("c")
        base = wid * tpw
        waits = []
        for c in range(nch):
            off = base + c * cs
            pltpu.sync_copy(p1_hbm.at[pl.ds(off, cs)], p1_v.at[c])
            pltpu.sync_copy(p2_hbm.at[pl.ds(off, cs)], p2_v.at[c])
            pltpu.sync_copy(x_hbm.at[pl.ds(off, cs)], rows_v.at[c])
            waits.append(
                pltpu.async_copy(rows_v.at[c], xs_hbm.at[p1_v.at[c]],
                                 sems[2 * c]))
            waits.append(
                pltpu.async_copy(rows_v.at[c], xs_hbm.at[p2_v.at[c]],
                                 sems[2 * c + 1]))
        for wdesc in waits:
            wdesc.wait()

    return disp


def _make_combine(t, d, ntot):
    tpw = t // _NW
    nch = 4
    cs = tpw // nch
    mesh = plsc.VectorSubcoreMesh(core_axis_name="c", subcore_axis_name="s")

    @functools.partial(
        pl.kernel,
        mesh=mesh,
        out_type=jax.ShapeDtypeStruct((t, d), jnp.float32),
        scratch_types=[
            pltpu.VMEM((nch, cs), jnp.int32),
            pltpu.VMEM((nch, cs), jnp.int32),
            pltpu.VMEM((tpw, _L), jnp.float32),
            pltpu.VMEM((2, cs, d), jnp.float32),
            pltpu.VMEM((2, cs, d), jnp.float32),
            pltpu.VMEM((2, cs, d), jnp.float32),
        ]
        + [pltpu.SemaphoreType.DMA] * 6,
    )
    def comb(ys_hbm, p1_hbm, p2_hbm, w1_hbm, y_hbm,
             p1_v, p2_v, w_v, a_v, b_v, o_v, *sems):
        sa, sb, so = sems[0:2], sems[2:4], sems[4:6]
        wid = lax.axis_index("s") * _SC_INFO.num_cores + lax.axis_index("c")
        base = wid * tpw
        pltpu.sync_copy(w1_hbm.at[pl.ds(base, tpw)], w_v)

        def issue(c):
            pb = c % 2
            off = base + c * cs
            pltpu.sync_copy(p1_hbm.at[pl.ds(off, cs)], p1_v.at[c])
            pltpu.sync_copy(p2_hbm.at[pl.ds(off, cs)], p2_v.at[c])
            return (pltpu.async_copy(ys_hbm.at[p1_v.at[c]], a_v.at[pb], sa[pb]),
                    pltpu.async_copy(ys_hbm.at[p2_v.at[c]], b_v.at[pb], sb[pb]))

        gathers = {0: issue(0)}
        owaits = {}
        for c in range(nch):
            pb = c % 2
            ca, cb = gathers.pop(c)
            ca.wait()
            cb.wait()
            if c + 1 < nch:
                gathers[c + 1] = issue(c + 1)
            if c >= 2:
                owaits.pop(c - 2).wait()

            def row_body(r, carry, c=c, pb=pb):
                w1s = w_v[c * cs + r, :]
                w2s = 1.0 - w1s
                for j in range(d // _L):
                    sl = pl.ds(j * _L, _L)
                    o_v[pb, r, sl] = a_v[pb, r, sl] * w1s + b_v[pb, r, sl] * w2s
                return carry

            lax.fori_loop(0, cs, row_body, 0)
            owaits[c] = pltpu.async_copy(
                o_v.at[pb], y_hbm.at[pl.ds(base + c * cs, cs)], so[pb])
        for wdesc in owaits.values():
            wdesc.wait()

    return comb


def kernel(x, gate_W, gate_b, W1, b1, W2, b2):
    x_shape = x.shape
    d = x_shape[-1]
    xf = x.reshape(-1, d)
    t = xf.shape[0]
    nt = (t * _K) // _BT + _E
    ntot = nt * _BT

    prob, pos1, pos2, w1n, w2n, te, act = _gating(xf, gate_W, gate_b, nt)
    p1 = pos1.reshape(t)
    p2 = pos2.reshape(t)

    xs = _make_dispatch(t, d, ntot)(xf, p1, p2)

    ys = _grouped_ffn(xs, te[:, 0], act[:, 0], W1, b1, W2, b2, nt, d)

    y = _make_combine(t, d, ntot)(ys, p1, p2, w1n)
    return (y.reshape(x_shape), prob)


# speedup vs baseline: 1.0228x; 1.0228x over previous
import functools

import jax
import jax.numpy as jnp
from jax import lax
from jax.experimental import pallas as pl
from jax.experimental.pallas import tpu as pltpu
from jax.experimental.pallas import tpu_sc as plsc

_E = 8
_K = 2
_BT = 256
_CH = 512

_SC_INFO = plsc.get_sparse_core_info()
_NW = _SC_INFO.num_cores * _SC_INFO.num_subcores
_L = _SC_INFO.num_lanes


def _gate_kernel(x_ref, gw_ref, gb_ref, prob_ref, pos1_ref, pos2_ref,
                 w1n_ref, w2n_ref, te_ref, act_ref, xb_ref, nt):
    t = x_ref.shape[0]
    d = x_ref.shape[1]
    x = x_ref[...]
    xb_ref[...] = pltpu.pack_elementwise(
        [x[:, : d // 2], x[:, d // 2:]], packed_dtype=jnp.bfloat16)
    logits = jnp.dot(x, gw_ref[...],
                     preferred_element_type=jnp.float32) + gb_ref[...]
    m = jnp.max(logits, axis=1, keepdims=True)
    p = jnp.exp(logits - m)
    prob = p / jnp.sum(p, axis=1, keepdims=True)
    prob_ref[...] = prob

    iota_e = lax.broadcasted_iota(jnp.int32, (t, _E), 1)
    m1 = jnp.max(prob, axis=1, keepdims=True)
    i1 = jnp.min(jnp.where(prob == m1, iota_e, _E), axis=1, keepdims=True)
    masked = jnp.where(iota_e == i1, -1.0, prob)
    m2 = jnp.max(masked, axis=1, keepdims=True)
    i2 = jnp.min(jnp.where(masked == m2, iota_e, _E), axis=1, keepdims=True)

    e21 = jnp.exp(m2 - m1)
    w1n_ref[...] = jnp.broadcast_to(1.0 / (1.0 + e21), w1n_ref.shape)
    w2n_ref[...] = jnp.broadcast_to(e21 / (1.0 + e21), w2n_ref.shape)

    oh1 = (iota_e == i1).astype(jnp.float32)
    oh2 = (iota_e == i2).astype(jnp.float32)
    rr = lax.broadcasted_iota(jnp.int32, (_CH, _CH), 0)
    cc = lax.broadcasted_iota(jnp.int32, (_CH, _CH), 1)
    ltri = (cc < rr).astype(jnp.float32)

    base = jnp.zeros((1, _E), jnp.float32)
    ranks = []
    for oh in (oh1, oh2):
        for c in range(t // _CH):
            blk = oh[c * _CH:(c + 1) * _CH]
            cum = jnp.dot(ltri, blk, preferred_element_type=jnp.float32) + base
            ranks.append(cum)
            base = base + jnp.sum(blk, axis=0, keepdims=True)
    rank1 = jnp.concatenate(ranks[: t // _CH], axis=0)
    rank2 = jnp.concatenate(ranks[t // _CH:], axis=0)

    counts = base
    padded = jnp.ceil(counts / _BT) * _BT
    er = lax.broadcasted_iota(jnp.int32, (_E, _E), 0)
    ec = lax.broadcasted_iota(jnp.int32, (_E, _E), 1)
    u8 = (er < ec).astype(jnp.float32)
    pad_off = jnp.dot(padded, u8, preferred_element_type=jnp.float32)

    pos1 = jnp.sum((pad_off + rank1) * oh1, axis=1, keepdims=True)
    pos2 = jnp.sum((pad_off + rank2) * oh2, axis=1, keepdims=True)
    pos1_ref[...] = pos1.astype(jnp.int32)
    pos2_ref[...] = pos2.astype(jnp.int32)

    pad_end = pad_off + padded
    ts = lax.broadcasted_iota(jnp.int32, (nt, 1), 0).astype(jnp.float32) * _BT
    te = jnp.sum((pad_end <= ts).astype(jnp.int32), axis=1, keepdims=True)
    te = jnp.minimum(te, _E - 1)
    te_ref[...] = te
    tile_iota = lax.broadcasted_iota(jnp.int32, (nt, _E), 1)
    pe = jnp.sum(jnp.where(tile_iota == te, pad_end, 0.0), axis=1,
                 keepdims=True)
    act_ref[...] = (ts < pe).astype(jnp.int32)


def _gating(xf, gate_W, gate_b, nt):
    t = xf.shape[0]
    f32, i32 = jnp.float32, jnp.int32
    out_shape = (
        jax.ShapeDtypeStruct((t, _E), f32),
        jax.ShapeDtypeStruct((t, 1), i32),
        jax.ShapeDtypeStruct((t, 1), i32),
        jax.ShapeDtypeStruct((t, _L), f32),
        jax.ShapeDtypeStruct((t, _L), f32),
        jax.ShapeDtypeStruct((nt, 1), i32),
        jax.ShapeDtypeStruct((nt, 1), i32),
        jax.ShapeDtypeStruct((t, xf.shape[1] // 2), i32),
    )
    return pl.pallas_call(
        functools.partial(_gate_kernel, nt=nt),
        out_shape=out_shape,
    )(xf, gate_W, gate_b.reshape(1, _E))


def _ffn_kernel(te_ref, act_ref, xs_ref, w1_ref, b1_ref, w2_ref, b2_ref,
                out_ref):
    i = pl.program_id(0)
    e = te_ref[i]

    @pl.when(act_ref[i] != 0)
    def _():
        d2 = xs_ref.shape[1]
        xi = xs_ref[...]
        xa = pltpu.unpack_elementwise(
            xi, index=0, packed_dtype=jnp.bfloat16, unpacked_dtype=jnp.float32)
        xb = pltpu.unpack_elementwise(
            xi, index=1, packed_dtype=jnp.bfloat16, unpacked_dtype=jnp.float32)
        w1 = w1_ref[0]
        h = (jnp.dot(xa, w1[:d2], preferred_element_type=jnp.float32)
             + jnp.dot(xb, w1[d2:], preferred_element_type=jnp.float32))
        h = jnp.maximum(h + b1_ref[e][None, :], 0.0)
        y = jnp.dot(h, w2_ref[0], preferred_element_type=jnp.float32)
        out_ref[...] = y + b2_ref[e][None, :]


def _grouped_ffn(xs, tile_expert, act, W1, b1, W2, b2, nt, d):
    grid_spec = pltpu.PrefetchScalarGridSpec(
        num_scalar_prefetch=2,
        grid=(nt,),
        in_specs=[
            pl.BlockSpec((_BT, d // 2), lambda i, te, act: (act[i] * i, 0)),
            pl.BlockSpec((1, d, d), lambda i, te, act: (te[i], 0, 0)),
            pl.BlockSpec((_E, d), lambda i, te, act: (0, 0)),
            pl.BlockSpec((1, d, d), lambda i, te, act: (te[i], 0, 0)),
            pl.BlockSpec((_E, d), lambda i, te, act: (0, 0)),
        ],
        out_specs=pl.BlockSpec(
            (_BT, d), lambda i, te, act: (jnp.where(act[i] != 0, i, nt), 0)),
    )
    return pl.pallas_call(
        _ffn_kernel,
        grid_spec=grid_spec,
        out_shape=jax.ShapeDtypeStruct(((nt + 1) * _BT, d), jnp.float32),
    )(tile_expert, act, xs, W1, b1, W2, b2)


def _make_dispatch(t, d, ntot):
    tpw = t // _NW
    mesh = plsc.VectorSubcoreMesh(core_axis_name="c", subcore_axis_name="s")

    nch = 2
    cs = tpw // nch

    d2 = d // 2

    @functools.partial(
        pl.kernel,
        mesh=mesh,
        out_type=jax.ShapeDtypeStruct((ntot, d2), jnp.int32),
        scratch_types=[
            pltpu.VMEM((nch, cs), jnp.int32),
            pltpu.VMEM((nch, cs), jnp.int32),
            pltpu.VMEM((nch, cs, d2), jnp.int32),
            pltpu.SemaphoreType.DMA,
        ]
        + [pltpu.SemaphoreType.DMA] * (2 * nch),
    )
    def disp(x_hbm, p1_hbm, p2_hbm, xs_hbm, p1_v, p2_v, rows_v, rsem, *sems):
        wid = lax.axis_index("s") * _SC_INFO.num_cores + lax.axis_index("c")
        base = wid * tpw
        waits = []
        for c in range(nch):
            off = base + c * cs
            pltpu.sync_copy(p1_hbm.at[pl.ds(off, cs)], p1_v.at[c])
            pltpu.sync_copy(p2_hbm.at[pl.ds(off, cs)], p2_v.at[c])
            pltpu.sync_copy(x_hbm.at[pl.ds(off, cs)], rows_v.at[c])
            waits.append(
                pltpu.async_copy(rows_v.at[c], xs_hbm.at[p1_v.at[c]],
                                 sems[2 * c]))
            waits.append(
                pltpu.async_copy(rows_v.at[c], xs_hbm.at[p2_v.at[c]],
                                 sems[2 * c + 1]))
        for wdesc in waits:
            wdesc.wait()

    return disp


def _make_combine(t, d, ntot):
    tpw = t // _NW
    nch = 4
    cs = tpw // nch
    mesh = plsc.VectorSubcoreMesh(core_axis_name="c", subcore_axis_name="s")

    @functools.partial(
        pl.kernel,
        mesh=mesh,
        out_type=jax.ShapeDtypeStruct((t, d), jnp.float32),
        scratch_types=[
            pltpu.VMEM((nch, cs), jnp.int32),
            pltpu.VMEM((nch, cs), jnp.int32),
            pltpu.VMEM((tpw, _L), jnp.float32),
            pltpu.VMEM((2, cs, d), jnp.float32),
            pltpu.VMEM((2, cs, d), jnp.float32),
            pltpu.VMEM((2, cs, d), jnp.float32),
        ]
        + [pltpu.SemaphoreType.DMA] * 6,
    )
    def comb(ys_hbm, p1_hbm, p2_hbm, w1_hbm, y_hbm,
             p1_v, p2_v, w_v, a_v, b_v, o_v, *sems):
        sa, sb, so = sems[0:2], sems[2:4], sems[4:6]
        wid = lax.axis_index("s") * _SC_INFO.num_cores + lax.axis_index("c")
        base = wid * tpw
        pltpu.sync_copy(w1_hbm.at[pl.ds(base, tpw)], w_v)

        def issue(c):
            pb = c % 2
            off = base + c * cs
            pltpu.sync_copy(p1_hbm.at[pl.ds(off, cs)], p1_v.at[c])
            pltpu.sync_copy(p2_hbm.at[pl.ds(off, cs)], p2_v.at[c])
            return (pltpu.async_copy(ys_hbm.at[p1_v.at[c]], a_v.at[pb], sa[pb]),
                    pltpu.async_copy(ys_hbm.at[p2_v.at[c]], b_v.at[pb], sb[pb]))

        gathers = {0: issue(0)}
        owaits = {}
        for c in range(nch):
            pb = c % 2
            ca, cb = gathers.pop(c)
            ca.wait()
            cb.wait()
            if c + 1 < nch:
                gathers[c + 1] = issue(c + 1)
            if c >= 2:
                owaits.pop(c - 2).wait()

            def row_body(r, carry, c=c, pb=pb):
                w1s = w_v[c * cs + r, :]
                w2s = 1.0 - w1s
                for j in range(d // _L):
                    sl = pl.ds(j * _L, _L)
                    o_v[pb, r, sl] = a_v[pb, r, sl] * w1s + b_v[pb, r, sl] * w2s
                return carry

            lax.fori_loop(0, cs, row_body, 0)
            owaits[c] = pltpu.async_copy(
                o_v.at[pb], y_hbm.at[pl.ds(base + c * cs, cs)], so[pb])
        for wdesc in owaits.values():
            wdesc.wait()

    return comb


def kernel(x, gate_W, gate_b, W1, b1, W2, b2):
    x_shape = x.shape
    d = x_shape[-1]
    xf = x.reshape(-1, d)
    t = xf.shape[0]
    nt = (t * _K) // _BT + _E
    ntot = nt * _BT

    prob, pos1, pos2, w1n, w2n, te, act, xb = _gating(xf, gate_W, gate_b, nt)
    p1 = pos1.reshape(t)
    p2 = pos2.reshape(t)

    xs = _make_dispatch(t, d, ntot)(xb, p1, p2)

    ys = _grouped_ffn(xs, te[:, 0], act[:, 0], W1, b1, W2, b2, nt, d)

    y = _make_combine(t, d, ntot)(ys, p1, p2, w1n)
    return (y.reshape(x_shape), prob)


# speedup vs baseline: 1.0327x; 1.0097x over previous
import functools

import jax
import jax.numpy as jnp
from jax import lax
from jax.experimental import pallas as pl
from jax.experimental.pallas import tpu as pltpu
from jax.experimental.pallas import tpu_sc as plsc

_E = 8
_K = 2
_BT = 256
_CH = 512

_SC_INFO = plsc.get_sparse_core_info()
_NW = _SC_INFO.num_cores * _SC_INFO.num_subcores
_L = _SC_INFO.num_lanes


def _gate_kernel(x_ref, gw_ref, gb_ref, prob_ref, pos1_ref, pos2_ref,
                 w1n_ref, w2n_ref, te_ref, act_ref, xb_ref, nt):
    t = x_ref.shape[0]
    d = x_ref.shape[1]
    x = x_ref[...]
    xb_ref[...] = pltpu.pack_elementwise(
        [x[:, : d // 2], x[:, d // 2:]], packed_dtype=jnp.bfloat16)
    logits = jnp.dot(x, gw_ref[...],
                     preferred_element_type=jnp.float32) + gb_ref[...]
    m = jnp.max(logits, axis=1, keepdims=True)
    p = jnp.exp(logits - m)
    prob = p / jnp.sum(p, axis=1, keepdims=True)
    prob_ref[...] = prob

    iota_e = lax.broadcasted_iota(jnp.int32, (t, _E), 1)
    m1 = jnp.max(prob, axis=1, keepdims=True)
    i1 = jnp.min(jnp.where(prob == m1, iota_e, _E), axis=1, keepdims=True)
    masked = jnp.where(iota_e == i1, -1.0, prob)
    m2 = jnp.max(masked, axis=1, keepdims=True)
    i2 = jnp.min(jnp.where(masked == m2, iota_e, _E), axis=1, keepdims=True)

    e21 = jnp.exp(m2 - m1)
    w1n_ref[...] = jnp.broadcast_to(1.0 / (1.0 + e21), w1n_ref.shape)
    w2n_ref[...] = jnp.broadcast_to(e21 / (1.0 + e21), w2n_ref.shape)

    oh1 = (iota_e == i1).astype(jnp.float32)
    oh2 = (iota_e == i2).astype(jnp.float32)
    rr = lax.broadcasted_iota(jnp.int32, (_CH, _CH), 0)
    cc = lax.broadcasted_iota(jnp.int32, (_CH, _CH), 1)
    ltri = (cc < rr).astype(jnp.float32)

    base = jnp.zeros((1, _E), jnp.float32)
    ranks = []
    for oh in (oh1, oh2):
        for c in range(t // _CH):
            blk = oh[c * _CH:(c + 1) * _CH]
            cum = jnp.dot(ltri, blk, preferred_element_type=jnp.float32) + base
            ranks.append(cum)
            base = base + jnp.sum(blk, axis=0, keepdims=True)
    rank1 = jnp.concatenate(ranks[: t // _CH], axis=0)
    rank2 = jnp.concatenate(ranks[t // _CH:], axis=0)

    counts = base
    padded = jnp.ceil(counts / _BT) * _BT
    er = lax.broadcasted_iota(jnp.int32, (_E, _E), 0)
    ec = lax.broadcasted_iota(jnp.int32, (_E, _E), 1)
    u8 = (er < ec).astype(jnp.float32)
    pad_off = jnp.dot(padded, u8, preferred_element_type=jnp.float32)

    pos1 = jnp.sum((pad_off + rank1) * oh1, axis=1, keepdims=True)
    pos2 = jnp.sum((pad_off + rank2) * oh2, axis=1, keepdims=True)
    pos1_ref[...] = pos1.astype(jnp.int32)
    pos2_ref[...] = pos2.astype(jnp.int32)

    pad_end = pad_off + padded
    ts = lax.broadcasted_iota(jnp.int32, (nt, 1), 0).astype(jnp.float32) * _BT
    te = jnp.sum((pad_end <= ts).astype(jnp.int32), axis=1, keepdims=True)
    te = jnp.minimum(te, _E - 1)
    te_ref[...] = te
    tile_iota = lax.broadcasted_iota(jnp.int32, (nt, _E), 1)
    pe = jnp.sum(jnp.where(tile_iota == te, pad_end, 0.0), axis=1,
                 keepdims=True)
    act_ref[...] = (ts < pe).astype(jnp.int32)


def _gating(xf, gate_W, gate_b, nt):
    t = xf.shape[0]
    f32, i32 = jnp.float32, jnp.int32
    out_shape = (
        jax.ShapeDtypeStruct((t, _E), f32),
        jax.ShapeDtypeStruct((t, 1), i32),
        jax.ShapeDtypeStruct((t, 1), i32),
        jax.ShapeDtypeStruct((t, _L), f32),
        jax.ShapeDtypeStruct((t, _L), f32),
        jax.ShapeDtypeStruct((nt, 1), i32),
        jax.ShapeDtypeStruct((nt, 1), i32),
        jax.ShapeDtypeStruct((t, xf.shape[1] // 2), i32),
    )
    return pl.pallas_call(
        functools.partial(_gate_kernel, nt=nt),
        out_shape=out_shape,
    )(xf, gate_W, gate_b.reshape(1, _E))


def _ffn_kernel(te_ref, act_ref, xs_ref, w1_ref, b1_ref, w2_ref, b2_ref,
                out_ref):
    i = pl.program_id(0)
    e = te_ref[i]

    @pl.when(act_ref[i] != 0)
    def _():
        d2 = xs_ref.shape[1]
        xi = xs_ref[...]
        xa = pltpu.unpack_elementwise(
            xi, index=0, packed_dtype=jnp.bfloat16, unpacked_dtype=jnp.float32)
        xb = pltpu.unpack_elementwise(
            xi, index=1, packed_dtype=jnp.bfloat16, unpacked_dtype=jnp.float32)
        w1 = w1_ref[0]
        h = (jnp.dot(xa, w1[:d2], preferred_element_type=jnp.float32)
             + jnp.dot(xb, w1[d2:], preferred_element_type=jnp.float32))
        h = jnp.maximum(h + b1_ref[e][None, :], 0.0)
        y = jnp.dot(h, w2_ref[0], preferred_element_type=jnp.float32)
        y = y + b2_ref[e][None, :]
        out_ref[...] = pltpu.pack_elementwise(
            [y[:, :d2], y[:, d2:]], packed_dtype=jnp.bfloat16)


def _grouped_ffn(xs, tile_expert, act, W1, b1, W2, b2, nt, d):
    grid_spec = pltpu.PrefetchScalarGridSpec(
        num_scalar_prefetch=2,
        grid=(nt,),
        in_specs=[
            pl.BlockSpec((_BT, d // 2), lambda i, te, act: (act[i] * i, 0)),
            pl.BlockSpec((1, d, d), lambda i, te, act: (te[i], 0, 0)),
            pl.BlockSpec((_E, d), lambda i, te, act: (0, 0)),
            pl.BlockSpec((1, d, d), lambda i, te, act: (te[i], 0, 0)),
            pl.BlockSpec((_E, d), lambda i, te, act: (0, 0)),
        ],
        out_specs=pl.BlockSpec(
            (_BT, d // 2), lambda i, te, act: (jnp.where(act[i] != 0, i, nt), 0)),
    )
    return pl.pallas_call(
        _ffn_kernel,
        grid_spec=grid_spec,
        out_shape=jax.ShapeDtypeStruct(((nt + 1) * _BT, d // 2), jnp.int32),
    )(tile_expert, act, xs, W1, b1, W2, b2)


def _make_dispatch(t, d, ntot):
    tpw = t // _NW
    mesh = plsc.VectorSubcoreMesh(core_axis_name="c", subcore_axis_name="s")

    nch = 2
    cs = tpw // nch

    d2 = d // 2

    @functools.partial(
        pl.kernel,
        mesh=mesh,
        out_type=jax.ShapeDtypeStruct((ntot, d2), jnp.int32),
        scratch_types=[
            pltpu.VMEM((nch, cs), jnp.int32),
            pltpu.VMEM((nch, cs), jnp.int32),
            pltpu.VMEM((nch, cs, d2), jnp.int32),
            pltpu.SemaphoreType.DMA,
        ]
        + [pltpu.SemaphoreType.DMA] * (2 * nch),
    )
    def disp(x_hbm, p1_hbm, p2_hbm, xs_hbm, p1_v, p2_v, rows_v, rsem, *sems):
        wid = lax.axis_index("s") * _SC_INFO.num_cores + lax.axis_index("c")
        base = wid * tpw
        waits = []
        for c in range(nch):
            off = base + c * cs
            pltpu.sync_copy(p1_hbm.at[pl.ds(off, cs)], p1_v.at[c])
            pltpu.sync_copy(p2_hbm.at[pl.ds(off, cs)], p2_v.at[c])
            pltpu.sync_copy(x_hbm.at[pl.ds(off, cs)], rows_v.at[c])
            waits.append(
                pltpu.async_copy(rows_v.at[c], xs_hbm.at[p1_v.at[c]],
                                 sems[2 * c]))
            waits.append(
                pltpu.async_copy(rows_v.at[c], xs_hbm.at[p2_v.at[c]],
                                 sems[2 * c + 1]))
        for wdesc in waits:
            wdesc.wait()

    return disp


def _make_combine(t, d, ntot):
    tpw = t // _NW
    nch = 4
    cs = tpw // nch
    d2 = d // 2
    mesh = plsc.VectorSubcoreMesh(core_axis_name="c", subcore_axis_name="s")

    @functools.partial(
        pl.kernel,
        mesh=mesh,
        out_type=jax.ShapeDtypeStruct((t, d), jnp.float32),
        scratch_types=[
            pltpu.VMEM((nch, cs), jnp.int32),
            pltpu.VMEM((nch, cs), jnp.int32),
            pltpu.VMEM((tpw, _L), jnp.float32),
            pltpu.VMEM((2, cs, d2), jnp.int32),
            pltpu.VMEM((2, cs, d2), jnp.int32),
            pltpu.VMEM((2, cs, d), jnp.float32),
        ]
        + [pltpu.SemaphoreType.DMA] * 6,
    )
    def comb(ys_hbm, p1_hbm, p2_hbm, w1_hbm, y_hbm,
             p1_v, p2_v, w_v, a_v, b_v, o_v, *sems):
        sa, sb, so = sems[0:2], sems[2:4], sems[4:6]
        wid = lax.axis_index("s") * _SC_INFO.num_cores + lax.axis_index("c")
        base = wid * tpw
        pltpu.sync_copy(w1_hbm.at[pl.ds(base, tpw)], w_v)

        def issue(c):
            pb = c % 2
            off = base + c * cs
            pltpu.sync_copy(p1_hbm.at[pl.ds(off, cs)], p1_v.at[c])
            pltpu.sync_copy(p2_hbm.at[pl.ds(off, cs)], p2_v.at[c])
            return (pltpu.async_copy(ys_hbm.at[p1_v.at[c]], a_v.at[pb], sa[pb]),
                    pltpu.async_copy(ys_hbm.at[p2_v.at[c]], b_v.at[pb], sb[pb]))

        gathers = {0: issue(0)}
        owaits = {}
        for c in range(nch):
            pb = c % 2
            ca, cb = gathers.pop(c)
            ca.wait()
            cb.wait()
            if c + 1 < nch:
                gathers[c + 1] = issue(c + 1)
            if c >= 2:
                owaits.pop(c - 2).wait()

            def row_body(r, carry, c=c, pb=pb):
                w1s = w_v[c * cs + r, :]
                w2s = 1.0 - w1s
                himask = jnp.int32(-65536)
                for j in range(d2 // _L):
                    sl = pl.ds(j * _L, _L)
                    a = a_v[pb, r, sl]
                    b = b_v[pb, r, sl]
                    alo = lax.bitcast_convert_type(a << 16, jnp.float32)
                    blo = lax.bitcast_convert_type(b << 16, jnp.float32)
                    ahi = lax.bitcast_convert_type(a & himask, jnp.float32)
                    bhi = lax.bitcast_convert_type(b & himask, jnp.float32)
                    o_v[pb, r, sl] = alo * w1s + blo * w2s
                    o_v[pb, r, pl.ds(d2 + j * _L, _L)] = ahi * w1s + bhi * w2s
                return carry

            lax.fori_loop(0, cs, row_body, 0)
            owaits[c] = pltpu.async_copy(
                o_v.at[pb], y_hbm.at[pl.ds(base + c * cs, cs)], so[pb])
        for wdesc in owaits.values():
            wdesc.wait()

    return comb


def kernel(x, gate_W, gate_b, W1, b1, W2, b2):
    x_shape = x.shape
    d = x_shape[-1]
    xf = x.reshape(-1, d)
    t = xf.shape[0]
    nt = (t * _K) // _BT + _E
    ntot = nt * _BT

    prob, pos1, pos2, w1n, w2n, te, act, xb = _gating(xf, gate_W, gate_b, nt)
    p1 = pos1.reshape(t)
    p2 = pos2.reshape(t)

    xs = _make_dispatch(t, d, ntot)(xb, p1, p2)

    ys = _grouped_ffn(xs, te[:, 0], act[:, 0], W1, b1, W2, b2, nt, d)

    y = _make_combine(t, d, ntot)(ys, p1, p2, w1n)
    return (y.reshape(x_shape), prob)
